# Initial kernel scaffold; baseline (speedup 1.0000x reference)
import functools

import jax
import jax.numpy as jnp
from jax import lax
from jax.experimental import pallas as pl
from jax.experimental.pallas import tpu as pltpu
from jax.experimental.pallas import tpu_sc as plsc

D = 128
G = 1024
NC = 2
NS = 16
NW = NC * NS
SPW = G // NW
NV = D // 16
CHUNK = 256
NEG_BIG = -3.0e38


def _tec_kernel(n_rows, x_hbm, b_hbm, wb_hbm, starts_hbm, out_hbm,
                xbuf, bbuf, wbuf, svbuf, sum_ref, max_ref, att_ref, obuf):
    cid = lax.axis_index("c")
    sid = lax.axis_index("s")
    w = cid * NS + sid
    seg_lo = w * SPW

    pltpu.sync_copy(wb_hbm, wbuf)
    pltpu.sync_copy(starts_hbm.at[pl.ds(seg_lo, 40)], svbuf)

    wv = [wbuf[pl.ds(16 * k, 16)] for k in range(NV)]
    bias = wbuf[pl.ds(D, 16)]

    zeros = jnp.zeros((16,), jnp.float32)
    neg = jnp.full((16,), NEG_BIG, jnp.float32)

    def zero_body(t, _):
        sum_ref[pl.ds(16 * t, 16)] = zeros
        att_ref[pl.ds(16 * t, 16)] = zeros
        max_ref[pl.ds(16 * t, 16)] = neg
        return 0

    lax.fori_loop(0, SPW * NV, zero_body, 0)

    rs = svbuf[0]
    re = svbuf[SPW]
    rs8 = (rs // 8) * 8
    nchunk = (re - rs8 + CHUNK - 1) // CHUNK

    def chunk_body(k, _):
        cbase = rs8 + k * CHUNK
        cstart = jnp.minimum(cbase, n_rows - CHUNK)
        pltpu.sync_copy(x_hbm.at[pl.ds(cstart, CHUNK)], xbuf)
        pltpu.sync_copy(b_hbm.at[pl.ds(cstart, CHUNK)], bbuf)

        def row_body(r, _):
            g = cstart + r
            b = bbuf[r]
            valid = (g >= cbase) & (b >= seg_lo) & (b < seg_lo + SPW)
            idx = jnp.where(valid, b - seg_lo, 0)
            base = idx * D
            xs = [xbuf[r, pl.ds(16 * k2, 16)] for k2 in range(NV)]
            t = xs[0] * wv[0]
            for k2 in range(1, NV):
                t = t + xs[k2] * wv[k2]
            z = jnp.sum(t)
            zv = jnp.full((16,), z, jnp.float32) + bias
            wgt = 1.0 / (1.0 + jnp.exp(-zv))
            vf = jnp.where(valid, 1.0, 0.0)
            wgt = wgt * vf
            for k2 in range(NV):
                off = base + 16 * k2
                xm = xs[k2] * vf
                sum_ref[pl.ds(off, 16)] = sum_ref[pl.ds(off, 16)] + xm
                att_ref[pl.ds(off, 16)] = att_ref[pl.ds(off, 16)] + xs[k2] * wgt
                xmax = jnp.where(valid, xs[k2], neg)
                max_ref[pl.ds(off, 16)] = jnp.maximum(max_ref[pl.ds(off, 16)], xmax)
            return 0

        lax.fori_loop(0, CHUNK, row_body, 0)
        return 0

    lax.fori_loop(0, nchunk, chunk_body, 0)

    def fin_body(s, _):
        cnt = (svbuf[s + 1] - svbuf[s]).astype(jnp.float32)
        inv = 1.0 / jnp.maximum(cnt, 1.0)
        has = cnt > 0.0
        for k2 in range(NV):
            sv = sum_ref[pl.ds(s * D + 16 * k2, 16)]
            obuf[s, pl.ds(16 * k2, 16)] = sv * inv
            mv = max_ref[pl.ds(s * D + 16 * k2, 16)]
            obuf[s, pl.ds(D + 16 * k2, 16)] = jnp.where(has, mv, zeros)
            av = att_ref[pl.ds(s * D + 16 * k2, 16)]
            obuf[s, pl.ds(2 * D + 16 * k2, 16)] = av
        return 0

    lax.fori_loop(0, SPW, fin_body, 0)
    pltpu.sync_copy(obuf, out_hbm.at[pl.ds(seg_lo, SPW)])


def kernel(x, batch, att_w, att_b):
    n = x.shape[0]
    batch32 = batch.astype(jnp.int32)
    bounds = jnp.arange(G + 1, dtype=jnp.int32)
    starts = jnp.searchsorted(batch32, bounds, side="left").astype(jnp.int32)
    starts = jnp.pad(starts, (0, 15))
    wb = jnp.concatenate(
        [att_w[:, 0], jnp.broadcast_to(att_b.astype(jnp.float32), (16,))]
    )

    mesh = plsc.VectorSubcoreMesh(core_axis_name="c", subcore_axis_name="s")
    run = pl.kernel(
        functools.partial(_tec_kernel, n),
        out_type=jax.ShapeDtypeStruct((G, 3 * D), jnp.float32),
        mesh=mesh,
        scratch_types=[
            pltpu.VMEM((CHUNK, D), jnp.float32),
            pltpu.VMEM((CHUNK,), jnp.int32),
            pltpu.VMEM((D + 16,), jnp.float32),
            pltpu.VMEM((40,), jnp.int32),
            pltpu.VMEM((SPW * D,), jnp.float32),
            pltpu.VMEM((SPW * D,), jnp.float32),
            pltpu.VMEM((SPW * D,), jnp.float32),
            pltpu.VMEM((SPW, 3 * D), jnp.float32),
        ],
    )
    return run(x, batch32, wb, starts)


# speedup vs baseline: 1.3757x; 1.3757x over previous
import functools

import jax
import jax.numpy as jnp
from jax import lax
from jax.experimental import pallas as pl
from jax.experimental.pallas import tpu as pltpu
from jax.experimental.pallas import tpu_sc as plsc

D = 128
G = 1024
NC = 2
NS = 16
NW = NC * NS
SPW = G // NW
NV = D // 16
CHUNK = 256
NEG_BIG = -3.0e38


def _tec_kernel(n_rows, x_hbm, b_hbm, wb_hbm, starts_hbm, out_hbm,
                xbuf, bbuf, wbuf, svbuf, sum_ref, max_ref, att_ref, obuf):
    cid = lax.axis_index("c")
    sid = lax.axis_index("s")
    w = cid * NS + sid
    seg_lo = w * SPW

    pltpu.sync_copy(wb_hbm, wbuf)
    pltpu.sync_copy(starts_hbm.at[pl.ds(seg_lo, 48)], svbuf)

    wv = [wbuf[pl.ds(16 * k, 16)] for k in range(NV)]
    bias = wbuf[pl.ds(D, 16)]

    zeros = jnp.zeros((16,), jnp.float32)
    neg = jnp.full((16,), NEG_BIG, jnp.float32)

    def zero_body(t, _):
        sum_ref[pl.ds(16 * t, 16)] = zeros
        att_ref[pl.ds(16 * t, 16)] = zeros
        max_ref[pl.ds(16 * t, 16)] = neg
        return 0

    lax.fori_loop(0, SPW * NV, zero_body, 0)

    sv0 = svbuf[pl.ds(0, 16)]
    sv1 = svbuf[pl.ds(16, 16)]
    sv2 = svbuf[pl.ds(32, 16)]
    rs = sv0[0]
    re = sv2[0]
    rs8 = (rs // 16) * 16
    nchunk = (re - rs8 + CHUNK - 1) // CHUNK

    def chunk_body(k, _):
        cbase = rs8 + k * CHUNK
        cstart = jnp.minimum(cbase, n_rows - CHUNK)
        pltpu.sync_copy(x_hbm.at[pl.ds(cstart, CHUNK)], xbuf)
        pltpu.sync_copy(b_hbm.at[pl.ds(cstart, CHUNK)], bbuf)

        def grp_body(q, _):
            bv = bbuf[pl.ds(16 * q, 16)]
            for j in range(16):
                r = 16 * q + j
                g = cstart + r
                b = bv[j]
                valid = (g >= cbase) & (b >= seg_lo) & (b < seg_lo + SPW)
                idx = jnp.where(valid, b - seg_lo, 0)
                base = idx * D
                xs = [xbuf[r, pl.ds(16 * k2, 16)] for k2 in range(NV)]
                t = xs[0] * wv[0]
                for k2 in range(1, NV):
                    t = t + xs[k2] * wv[k2]
                z = jnp.sum(t)
                zv = jnp.full((16,), z, jnp.float32) + bias
                wgt = 1.0 / (1.0 + jnp.exp(-zv))
                vf = jnp.where(valid, 1.0, 0.0)
                wgt = wgt * vf
                for k2 in range(NV):
                    off = base + 16 * k2
                    xm = xs[k2] * vf
                    sum_ref[pl.ds(off, 16)] = sum_ref[pl.ds(off, 16)] + xm
                    att_ref[pl.ds(off, 16)] = att_ref[pl.ds(off, 16)] + xs[k2] * wgt
                    xmax = jnp.where(valid, xs[k2], neg)
                    max_ref[pl.ds(off, 16)] = jnp.maximum(
                        max_ref[pl.ds(off, 16)], xmax)
            return 0

        lax.fori_loop(0, CHUNK // 16, grp_body, 0)
        return 0

    lax.fori_loop(0, nchunk, chunk_body, 0)

    svs = [sv0, sv1, sv2]
    for s in range(SPW):
        lo = svs[s // 16][s % 16]
        s1 = s + 1
        hi = svs[s1 // 16][s1 % 16]
        cnt = hi - lo
        cntv = jnp.full((16,), cnt, jnp.int32).astype(jnp.float32)
        inv = 1.0 / jnp.maximum(cntv, 1.0)
        has = cnt > 0
        for k2 in range(NV):
            sv = sum_ref[pl.ds(s * D + 16 * k2, 16)]
            obuf[s, pl.ds(16 * k2, 16)] = sv * inv
            mv = max_ref[pl.ds(s * D + 16 * k2, 16)]
            obuf[s, pl.ds(D + 16 * k2, 16)] = jnp.where(has, mv, zeros)
            av = att_ref[pl.ds(s * D + 16 * k2, 16)]
            obuf[s, pl.ds(2 * D + 16 * k2, 16)] = av
    pltpu.sync_copy(obuf, out_hbm.at[pl.ds(seg_lo, SPW)])


def kernel(x, batch, att_w, att_b):
    n = x.shape[0]
    batch32 = batch.astype(jnp.int32)
    bounds = jnp.arange(G + 1, dtype=jnp.int32)
    starts = jnp.searchsorted(batch32, bounds, side="left").astype(jnp.int32)
    starts = jnp.pad(starts, (0, 23))
    wb = jnp.concatenate(
        [att_w[:, 0], jnp.broadcast_to(att_b.astype(jnp.float32), (16,))]
    )

    mesh = plsc.VectorSubcoreMesh(core_axis_name="c", subcore_axis_name="s")
    run = pl.kernel(
        functools.partial(_tec_kernel, n),
        out_type=jax.ShapeDtypeStruct((G, 3 * D), jnp.float32),
        mesh=mesh,
        compiler_params=pltpu.CompilerParams(needs_layout_passes=False),
        scratch_types=[
            pltpu.VMEM((CHUNK, D), jnp.float32),
            pltpu.VMEM((CHUNK,), jnp.int32),
            pltpu.VMEM((D + 16,), jnp.float32),
            pltpu.VMEM((48,), jnp.int32),
            pltpu.VMEM((SPW * D,), jnp.float32),
            pltpu.VMEM((SPW * D,), jnp.float32),
            pltpu.VMEM((SPW * D,), jnp.float32),
            pltpu.VMEM((SPW, 3 * D), jnp.float32),
        ],
    )
    return run(x, batch32, wb, starts)


# speedup vs baseline: 4.3402x; 3.1548x over previous
import functools

import jax
import jax.numpy as jnp
from jax import lax
from jax.experimental import pallas as pl
from jax.experimental.pallas import tpu as pltpu
from jax.experimental.pallas import tpu_sc as plsc

D = 128
G = 1024
NC = 2
NS = 16
NW = NC * NS
SPW = G // NW
NV = D // 16
CHUNK = 256
NEG_BIG = -3.0e38


def _tec_kernel(n_rows, x_hbm, wb_hbm, starts_hbm, out_hbm,
                xbuf, wbuf, svbuf, ssm, sum_ref, max_ref, att_ref, obuf):
    cid = lax.axis_index("c")
    sid = lax.axis_index("s")
    w = cid * NS + sid
    seg_lo = w * SPW

    pltpu.sync_copy(wb_hbm, wbuf)
    pltpu.sync_copy(starts_hbm.at[pl.ds(seg_lo, 48)], svbuf)
    svs = [svbuf[pl.ds(16 * i, 16)] for i in range(3)]
    for i in range(SPW + 1):
        ssm[i] = svs[i // 16][i % 16]

    wv = [wbuf[pl.ds(16 * k, 16)] for k in range(NV)]
    bias = wbuf[pl.ds(D, 16)]

    zeros = jnp.zeros((16,), jnp.float32)
    neg = jnp.full((16,), NEG_BIG, jnp.float32)

    def zero_body(t, _):
        sum_ref[pl.ds(16 * t, 16)] = zeros
        att_ref[pl.ds(16 * t, 16)] = zeros
        max_ref[pl.ds(16 * t, 16)] = neg
        return 0

    lax.fori_loop(0, SPW * NV, zero_body, 0)

    rs = ssm[0]
    re = ssm[SPW]
    rs16 = (rs // 16) * 16
    nchunk = (re - rs16 + CHUNK - 1) // CHUNK

    init = (tuple(zeros for _ in range(NV)),
            tuple(zeros for _ in range(NV)),
            tuple(neg for _ in range(NV)))

    def chunk_body(k, _):
        cbase = rs16 + k * CHUNK
        cstart = jnp.minimum(cbase, n_rows - CHUNK)
        cend = cstart + CHUNK
        pltpu.sync_copy(x_hbm.at[pl.ds(cstart, CHUNK)], xbuf)

        def seg_body(s, _):
            lo = jnp.maximum(ssm[s], cbase)
            hi = jnp.minimum(ssm[s + 1], cend)

            @pl.when(lo < hi)
            def _():
                def row_body(r, carry):
                    sums, atts, mxs = carry
                    rl = r - cstart
                    xs = [xbuf[rl, pl.ds(16 * k2, 16)] for k2 in range(NV)]
                    t = xs[0] * wv[0]
                    for k2 in range(1, NV):
                        t = t + xs[k2] * wv[k2]
                    z = jnp.sum(t)
                    zv = jnp.full((16,), z, jnp.float32) + bias
                    wgt = 1.0 / (1.0 + jnp.exp(-zv))
                    sums = tuple(sums[k2] + xs[k2] for k2 in range(NV))
                    atts = tuple(atts[k2] + xs[k2] * wgt for k2 in range(NV))
                    mxs = tuple(jnp.maximum(mxs[k2], xs[k2])
                                for k2 in range(NV))
                    return (sums, atts, mxs)

                sums, atts, mxs = lax.fori_loop(lo, hi, row_body, init)
                for k2 in range(NV):
                    off = s * D + 16 * k2
                    sum_ref[pl.ds(off, 16)] = sum_ref[pl.ds(off, 16)] + sums[k2]
                    att_ref[pl.ds(off, 16)] = att_ref[pl.ds(off, 16)] + atts[k2]
                    max_ref[pl.ds(off, 16)] = jnp.maximum(
                        max_ref[pl.ds(off, 16)], mxs[k2])

            return 0

        lax.fori_loop(0, SPW, seg_body, 0)
        return 0

    lax.fori_loop(0, nchunk, chunk_body, 0)

    def fin_body(s, _):
        cnt = ssm[s + 1] - ssm[s]
        cntv = jnp.full((16,), cnt, jnp.int32).astype(jnp.float32)
        inv = 1.0 / jnp.maximum(cntv, 1.0)
        has = cnt > 0
        for k2 in range(NV):
            sv = sum_ref[pl.ds(s * D + 16 * k2, 16)]
            obuf[s, pl.ds(16 * k2, 16)] = sv * inv
            mv = max_ref[pl.ds(s * D + 16 * k2, 16)]
            obuf[s, pl.ds(D + 16 * k2, 16)] = jnp.where(has, mv, zeros)
            av = att_ref[pl.ds(s * D + 16 * k2, 16)]
            obuf[s, pl.ds(2 * D + 16 * k2, 16)] = av
        return 0

    lax.fori_loop(0, SPW, fin_body, 0)
    pltpu.sync_copy(obuf, out_hbm.at[pl.ds(seg_lo, SPW)])


def kernel(x, batch, att_w, att_b):
    n = x.shape[0]
    batch32 = batch.astype(jnp.int32)
    bounds = jnp.arange(G + 1, dtype=jnp.int32)
    starts = jnp.searchsorted(batch32, bounds, side="left").astype(jnp.int32)
    starts = jnp.pad(starts, (0, 23))
    wb = jnp.concatenate(
        [att_w[:, 0], jnp.broadcast_to(att_b.astype(jnp.float32), (16,))]
    )

    mesh = plsc.VectorSubcoreMesh(core_axis_name="c", subcore_axis_name="s")
    run = pl.kernel(
        functools.partial(_tec_kernel, n),
        out_type=jax.ShapeDtypeStruct((G, 3 * D), jnp.float32),
        mesh=mesh,
        compiler_params=pltpu.CompilerParams(needs_layout_passes=False),
        scratch_types=[
            pltpu.VMEM((CHUNK, D), jnp.float32),
            pltpu.VMEM((D + 16,), jnp.float32),
            pltpu.VMEM((48,), jnp.int32),
            pltpu.SMEM((48,), jnp.int32),
            pltpu.VMEM((SPW * D,), jnp.float32),
            pltpu.VMEM((SPW * D,), jnp.float32),
            pltpu.VMEM((SPW * D,), jnp.float32),
            pltpu.VMEM((SPW, 3 * D), jnp.float32),
        ],
    )
    return run(x, wb, starts)
